# baseline (speedup 1.0000x reference)
import functools

import jax
import jax.numpy as jnp
from jax import lax
from jax.experimental import pallas as pl
from jax.experimental.pallas import tpu as pltpu
from jax.experimental.pallas import tpu_sc as plsc

NN = 10000
EE = 320000
DD = 128
NC = 2
NS = 16
NW = NC * NS
EPW = EE // NW
CH = 80
NCHUNK = EPW // CH
STRIDE = 624
NCOPY = 8
HWORDS = 10000
_F32 = jnp.float32


def _ln(x, g, b):
    m = jnp.mean(x, axis=-1, keepdims=True)
    v = jnp.mean((x - m) * (x - m), axis=-1, keepdims=True)
    return (x - m) / jnp.sqrt(v + 1e-5) * g + b


def _tc_a(x, Wi, bi, g0, b0):
    blk = 1000

    def body(x_ref, w_ref, b_ref, g_ref, bb_ref, o_ref):
        h = jnp.dot(x_ref[...], w_ref[...], preferred_element_type=_F32)
        h = h + b_ref[...]
        o_ref[...] = jax.nn.gelu(_ln(h, g_ref[...], bb_ref[...]))

    full = pl.BlockSpec((DD, DD), lambda i: (0, 0))
    vec = pl.BlockSpec((1, DD), lambda i: (0, 0))
    return pl.pallas_call(
        body,
        grid=(NN // blk,),
        in_specs=[pl.BlockSpec((blk, DD), lambda i: (i, 0)), full, vec, vec, vec],
        out_specs=pl.BlockSpec((blk, DD), lambda i: (i, 0)),
        out_shape=jax.ShapeDtypeStruct((NN, DD), _F32),
    )(x, Wi, bi, g0, b0)


_GDN = lax.GatherDimensionNumbers(
    offset_dims=(), collapsed_slice_dims=(0,), start_index_map=(0,))


def _vgather(v, idx):
    return lax.gather(v, idx, _GDN, (1,),
                      mode=lax.GatherScatterMode.PROMISE_IN_BOUNDS)


def _zero_buf(buf, nrows):
    def zrow(r, _):
        for c in range(DD // 16):
            buf[r, pl.ds(c * 16, 16)] = jnp.zeros((16,), _F32)
        return 0
    lax.fori_loop(0, nrows, zrow, 0)


def _zero_hist(hist):
    def zv(i, _):
        hist[pl.ds(i * 16, 16)] = jnp.zeros((16,), _F32)
        return 0
    lax.fori_loop(0, HWORDS // 16, zv, 0)


def _zero_acc_stripe(rows, acc, sid):
    base_r = sid * STRIDE
    for j in range(NCOPY):
        pltpu.sync_copy(rows, acc.at[pl.ds(base_r + j * CH, CH)])


def _dump_acc_stripe(rows, acc, out_hbm, cid, sid):
    base_r = sid * STRIDE
    for j in range(NCOPY):
        off = base_r + j * CH
        pltpu.sync_copy(acc.at[pl.ds(off, CH)], rows)
        pltpu.sync_copy(rows, out_hbm.at[cid, pl.ds(off, CH)])


def _sc_sage(h0, src, dst):
    mesh = plsc.VectorSubcoreMesh(core_axis_name="c", subcore_axis_name="s")

    @functools.partial(
        pl.kernel,
        out_type=(jax.ShapeDtypeStruct((NC, NN, DD), _F32),
                  jax.ShapeDtypeStruct((NW, HWORDS), _F32)),
        mesh=mesh,
        compiler_params=pltpu.CompilerParams(needs_layout_passes=False),
        scratch_types=[
            pltpu.VMEM((CH,), jnp.int32),
            pltpu.VMEM((CH,), jnp.int32),
            pltpu.VMEM((CH,), jnp.int32),
            pltpu.VMEM((CH,), jnp.int32),
            pltpu.VMEM((CH, DD), _F32),
            pltpu.VMEM((CH, DD), _F32),
            pltpu.VMEM((HWORDS,), _F32),
            pltpu.VMEM_SHARED((NN, DD), _F32),
            pltpu.SemaphoreType.DMA,
            pltpu.SemaphoreType.DMA,
        ],
    )
    def k(h0_hbm, src_hbm, dst_hbm, out_hbm, deg_hbm,
          idx_sA, idx_dA, idx_sB, idx_dB, rowsA, rowsB, hist, acc,
          semA, semB):
        cid = lax.axis_index("c")
        sid = lax.axis_index("s")
        wid = cid * NS + sid
        iot = lax.iota(jnp.int32, 16)
        lmask = [iot == j for j in range(16)]
        ones_v = jnp.full((16,), 1.0, _F32)

        _zero_buf(rowsA, CH)
        _zero_hist(hist)
        _zero_acc_stripe(rowsA, acc, sid)
        plsc.subcore_barrier()

        def load_idx(i, bs, bd):
            base = wid * EPW + i * CH
            pltpu.sync_copy(src_hbm.at[pl.ds(base, CH)], bs)
            pltpu.sync_copy(dst_hbm.at[pl.ds(base, CH)], bd)

        def deg_pass(bd):
            @plsc.parallel_loop(0, CH // 16)
            def deg_grp(g):
                dvec = bd[pl.ds(g * 16, 16)]
                for j in range(16):
                    plsc.addupdate_scatter(hist, [dvec], ones_v, mask=lmask[j])

        load_idx(0, idx_sA, idx_dA)
        pltpu.async_copy(h0_hbm.at[idx_sA], rowsA, semA)

        def pair(g, _):
            load_idx(2 * g + 1, idx_sB, idx_dB)
            pltpu.async_copy(h0_hbm.at[idx_sB], rowsB, semB)
            pltpu.make_async_copy(h0_hbm.at[idx_sA], rowsA, semA).wait()
            deg_pass(idx_dA)
            pltpu.sync_copy(rowsA, acc.at[idx_dA], add=True)
            load_idx(2 * g + 2, idx_sA, idx_dA)
            pltpu.async_copy(h0_hbm.at[idx_sA], rowsA, semA)
            pltpu.make_async_copy(h0_hbm.at[idx_sB], rowsB, semB).wait()
            deg_pass(idx_dB)
            pltpu.sync_copy(rowsB, acc.at[idx_dB], add=True)
            return 0

        lax.fori_loop(0, (NCHUNK - 1) // 2, pair, 0)
        pltpu.make_async_copy(h0_hbm.at[idx_sA], rowsA, semA).wait()
        deg_pass(idx_dA)
        pltpu.sync_copy(rowsA, acc.at[idx_dA], add=True)
        plsc.subcore_barrier()

        _dump_acc_stripe(rowsA, acc, out_hbm, cid, sid)
        pltpu.sync_copy(hist, deg_hbm.at[wid])

    return k(h0, src, dst)


def _tc_c(p0, p1, degs, h0, s_Wl, s_bl, s_Wr, g1, b1,
          a_Wl, a_bl, a_Wr, a_br):
    blk = 1000

    def body(p0_ref, p1_ref, d_ref, h0_ref, swl, sbl, swr, g1r, b1r,
             awl, abl, awr, abr, h1_ref, xl_ref, xr_ref):
        deg = jnp.sum(d_ref[...], axis=-1, keepdims=True)
        agg = (p0_ref[...] + p1_ref[...]) / jnp.maximum(deg, 1.0)
        h0v = h0_ref[...]
        h1 = (jnp.dot(agg, swl[...], preferred_element_type=_F32) + sbl[...]
              + jnp.dot(h0v, swr[...], preferred_element_type=_F32))
        h1 = jax.nn.gelu(_ln(h1, g1r[...], b1r[...])) + h0v
        h1_ref[...] = h1
        xl_ref[...] = jnp.dot(h1, awl[...], preferred_element_type=_F32) + abl[...]
        xr_ref[...] = jnp.dot(h1, awr[...], preferred_element_type=_F32) + abr[...]

    full = pl.BlockSpec((DD, DD), lambda i: (0, 0))
    vec = pl.BlockSpec((1, DD), lambda i: (0, 0))
    rows_d = pl.BlockSpec((blk, DD), lambda i: (i, 0))
    hcol = pl.BlockSpec((blk, DD), lambda i: (i, 0))
    return pl.pallas_call(
        body,
        grid=(NN // blk,),
        in_specs=[rows_d, rows_d, hcol, rows_d, full, vec, full, vec, vec,
                  full, vec, full, vec],
        out_specs=(rows_d, rows_d, rows_d),
        out_shape=(jax.ShapeDtypeStruct((NN, DD), _F32),
                   jax.ShapeDtypeStruct((NN, DD), _F32),
                   jax.ShapeDtypeStruct((NN, DD), _F32)),
    )(p0, p1, degs, h0, s_Wl, s_bl, s_Wr, g1, b1, a_Wl, a_bl, a_Wr, a_br)


def _sc_gat(xl, xr, att, src, dst):
    mesh = plsc.VectorSubcoreMesh(core_axis_name="c", subcore_axis_name="s")

    @functools.partial(
        pl.kernel,
        out_type=(jax.ShapeDtypeStruct((NC, NN, DD), _F32),
                  jax.ShapeDtypeStruct((NW, HWORDS), _F32)),
        mesh=mesh,
        compiler_params=pltpu.CompilerParams(needs_layout_passes=False),
        scratch_types=[
            pltpu.VMEM((CH,), jnp.int32),
            pltpu.VMEM((CH,), jnp.int32),
            pltpu.VMEM((CH,), jnp.int32),
            pltpu.VMEM((CH,), jnp.int32),
            pltpu.VMEM((CH, DD), _F32),
            pltpu.VMEM((CH, DD), _F32),
            pltpu.VMEM((CH, DD), _F32),
            pltpu.VMEM((HWORDS,), _F32),
            pltpu.VMEM((DD,), _F32),
            pltpu.VMEM_SHARED((NN, DD), _F32),
            pltpu.SemaphoreType.DMA,
            pltpu.SemaphoreType.DMA,
        ],
    )
    def k(xl_hbm, xr_hbm, att_hbm, src_hbm, dst_hbm, out_hbm, den_hbm,
          idx_sA, idx_dA, idx_sB, idx_dB, xlbA, xlbB, xrb,
          hist, attb, acc, semA, semB):
        cid = lax.axis_index("c")
        sid = lax.axis_index("s")
        wid = cid * NS + sid

        pltpu.sync_copy(att_hbm, attb)
        att_v = [attb[pl.ds(c * 16, 16)] for c in range(DD // 16)]
        iot = lax.iota(jnp.int32, 16)
        pidx = [(iot ^ (1 << kk))[:, None] for kk in range(4)]
        bmask = [((iot >> kk) & 1) == 1 for kk in range(4)]
        lmask = [iot == j for j in range(16)]
        cj = [(iot * 0 + j)[:, None] for j in range(16)]

        _zero_buf(xlbA, CH)
        _zero_hist(hist)
        _zero_acc_stripe(xlbA, acc, sid)
        plsc.subcore_barrier()

        def load_idx(i, bs, bd):
            base = wid * EPW + i * CH
            pltpu.sync_copy(src_hbm.at[pl.ds(base, CH)], bs)
            pltpu.sync_copy(dst_hbm.at[pl.ds(base, CH)], bd)

        def drain(bs, bd, xlb, sem):
            pltpu.make_async_copy(xl_hbm.at[bs], xlb, sem).wait()
            pltpu.make_async_copy(xr_hbm.at[bd], xrb, sem).wait()


        def compute(xlb, bd):
            @plsc.parallel_loop(0, CH // 16)
            def edge_grp(g):
                dvec = bd[pl.ds(g * 16, 16)]
                vecs = []
                for j in range(16):
                    e = g * 16 + j
                    acc_v = jnp.zeros((16,), _F32)
                    for c in range(DD // 16):
                        z = xlb[e, pl.ds(c * 16, 16)] + xrb[e, pl.ds(c * 16, 16)]
                        acc_v = acc_v + jnp.maximum(z, 0.2 * z) * att_v[c]
                    vecs.append(acc_v)
                for kk in range(4):
                    nxt = []
                    for p in range(0, len(vecs), 2):
                        a, b = vecs[p], vecs[p + 1]
                        s1 = a + _vgather(a, pidx[kk])
                        s2 = b + _vgather(b, pidx[kk])
                        nxt.append(jnp.where(bmask[kk], s2, s1))
                    vecs = nxt
                wexp = jnp.exp(vecs[0])
                for j in range(16):
                    e = g * 16 + j
                    wv = _vgather(wexp, cj[j])
                    for c in range(DD // 16):
                        xlb[e, pl.ds(c * 16, 16)] = xlb[e, pl.ds(c * 16, 16)] * wv
                    plsc.addupdate_scatter(hist, [dvec], wexp, mask=lmask[j])

        load_idx(0, idx_sA, idx_dA)
        pltpu.async_copy(xl_hbm.at[idx_sA], xlbA, semA)
        pltpu.async_copy(xr_hbm.at[idx_dA], xrb, semA)

        def pair(g, _):
            load_idx(2 * g + 1, idx_sB, idx_dB)
            pltpu.async_copy(xl_hbm.at[idx_sB], xlbB, semB)
            drain(idx_sA, idx_dA, xlbA, semA)
            compute(xlbA, idx_dA)
            pltpu.sync_copy(xlbA, acc.at[idx_dA], add=True)
            pltpu.async_copy(xr_hbm.at[idx_dB], xrb, semB)
            load_idx(2 * g + 2, idx_sA, idx_dA)
            pltpu.async_copy(xl_hbm.at[idx_sA], xlbA, semA)
            drain(idx_sB, idx_dB, xlbB, semB)
            compute(xlbB, idx_dB)
            pltpu.sync_copy(xlbB, acc.at[idx_dB], add=True)
            pltpu.async_copy(xr_hbm.at[idx_dA], xrb, semA)
            return 0

        lax.fori_loop(0, (NCHUNK - 1) // 2, pair, 0)
        drain(idx_sA, idx_dA, xlbA, semA)
        compute(xlbA, idx_dA)
        pltpu.sync_copy(xlbA, acc.at[idx_dA], add=True)
        plsc.subcore_barrier()

        _dump_acc_stripe(xlbA, acc, out_hbm, cid, sid)
        pltpu.sync_copy(hist, den_hbm.at[wid])

    return k(xl, xr, att, src, dst)


def _tc_e(q0, q1, dens, xl, xr, h1, att, a_bias, g2, b2,
          o_W1, o_b1, og, ob, o_W2, o_b2):
    blk = 1000

    def body(q0_ref, q1_ref, d_ref, xl_ref, xr_ref, h1_ref, att_ref,
             abias, g2r, b2r, w1, bb1, ogr, obr, w2, bb2, o_ref):
        xlv = xl_ref[...]
        z = xlv + xr_ref[...]
        lrelu = jnp.maximum(z, 0.2 * z)
        logit_self = jnp.sum(lrelu * att_ref[...], axis=-1, keepdims=True)
        wself = jnp.exp(logit_self)
        num = q0_ref[...] + q1_ref[...] + wself * xlv
        den = jnp.sum(d_ref[...], axis=-1, keepdims=True) + wself
        gat = num / den + abias[...]
        h2 = jax.nn.gelu(_ln(gat, g2r[...], b2r[...])) + h1_ref[...]
        t = jax.nn.gelu(_ln(jnp.dot(h2, w1[...], preferred_element_type=_F32)
                            + bb1[...], ogr[...], obr[...]))
        zf = jnp.dot(t, w2[...], preferred_element_type=_F32) + bb2[...]
        nrm = jnp.sqrt(jnp.sum(zf * zf, axis=-1, keepdims=True))
        o_ref[...] = zf / jnp.maximum(nrm, 1e-12)

    full = pl.BlockSpec((DD, DD), lambda i: (0, 0))
    vec = pl.BlockSpec((1, DD), lambda i: (0, 0))
    rows_d = pl.BlockSpec((blk, DD), lambda i: (i, 0))
    hcol = pl.BlockSpec((blk, DD), lambda i: (i, 0))
    return pl.pallas_call(
        body,
        grid=(NN // blk,),
        in_specs=[rows_d, rows_d, hcol, rows_d, rows_d, rows_d, vec, vec,
                  vec, vec, full, vec, vec, vec, full, vec],
        out_specs=rows_d,
        out_shape=jax.ShapeDtypeStruct((NN, DD), _F32),
    )(q0, q1, dens, xl, xr, h1, att, a_bias, g2, b2,
      o_W1, o_b1, og, ob, o_W2, o_b2)


def kernel(x, edge_index, Wi, bi, g0, b0, s_Wl, s_bl, s_Wr, g1, b1,
           a_Wl, a_bl, a_Wr, a_br, a_att, a_bias, g2, b2,
           o_W1, o_b1, og, ob, o_W2, o_b2):
    src = edge_index[0]
    dst = edge_index[1]
    r = lambda v: v.reshape(1, DD)

    hT = lambda h: jnp.pad(h.T, ((0, 0), (0, DD - NW)))

    h0 = _tc_a(x, Wi, r(bi), r(g0), r(b0))

    p, degp = _sc_sage(h0, src, dst)

    h1, xl, xr = _tc_c(p[0], p[1], hT(degp), h0, s_Wl, r(s_bl), s_Wr,
                       r(g1), r(b1), a_Wl, r(a_bl), a_Wr, r(a_br))

    q, denp = _sc_gat(xl, xr, a_att, src, dst)

    return _tc_e(q[0], q[1], hT(denp), xl, xr, h1, r(a_att), r(a_bias),
                 r(g2), r(b2), o_W1, r(o_b1), r(og), r(ob), o_W2, r(o_b2))
